# Initial kernel scaffold; baseline (speedup 1.0000x reference)
import jax
import jax.numpy as jnp
from jax.experimental import pallas as pl


def kernel(conv_w, conv_b, w_ih1, w_hh1, b_ih1, b_hh1, w_ih2, w_hh2, b_ih2, b_hh2, fc_w, fc_b, x):
    raise NotImplementedError("write your pallas kernel here")


# speedup vs baseline: 6.6145x; 6.6145x over previous
import jax
import jax.numpy as jnp
from jax import lax
from jax.experimental import pallas as pl
from jax.experimental.pallas import tpu as pltpu

H = 64
H2 = 2 * H
KW = 8
C_IN = 2
CK = C_IN * KW
CK2 = 2 * CK
L_IN = 128
T_CONV = L_IN - KW + 1
T = T_CONV // 2
NC = 11
NCP = 128
B_TILE = 1024
CONV_CHUNKS = 4


def _make_body(bh):
    R2 = T * bh
    CR = R2 // CONV_CHUNKS

    def body(p_ref, wconv_ref, bconv_ref, wih1_ref, whh1_ref, b1_ref,
             wih2_ref, whh2_ref, b2_ref, wfc_ref, bfc_ref,
             out_ref, pooled_ref):
        f32 = jnp.float32

        wconv = wconv_ref[...]
        bconv = bconv_ref[...]
        dn = (((0,), (0,)), ((), ()))
        for c in range(CONV_CHUNKS):
            r0 = c * CR
            pe = p_ref[0, :, pl.ds(r0, CR)]
            po = p_ref[0, :, pl.ds(R2 + r0, CR)]
            ce = jnp.maximum(
                lax.dot_general(pe, wconv, dn, preferred_element_type=f32)
                + bconv, 0.0)
            co = jnp.maximum(
                lax.dot_general(po, wconv, dn, preferred_element_type=f32)
                + bconv, 0.0)
            pooled_ref[pl.ds(r0, CR), :] = jnp.maximum(ce, co)

        wih1 = wih1_ref[...]
        whh1 = whh1_ref[...]
        bb1 = b1_ref[...]
        wih2 = wih2_ref[...]
        whh2 = whh2_ref[...]
        bb2 = b2_ref[...]
        bfc = bfc_ref[...]
        zeros = jnp.zeros((bh, H2), f32)

        def cell(gates, c):
            sig = jax.nn.sigmoid(gates[:, 0:3 * H2])
            g = jnp.tanh(gates[:, 3 * H2:4 * H2])
            c = sig[:, H2:2 * H2] * c + sig[:, 0:H2] * g
            h = sig[:, 2 * H2:3 * H2] * jnp.tanh(c)
            return h, c

        def step1(t, h1, c1):
            r0 = pl.multiple_of(t * bh, bh)
            gates = (jnp.dot(pooled_ref[pl.ds(r0, bh), :], wih1,
                             preferred_element_type=f32) + bb1
                     + jnp.dot(h1, whh1, preferred_element_type=f32))
            return cell(gates, c1)

        def step2(h1_prev, h2, c2):
            gates = (jnp.dot(h1_prev, wih2, preferred_element_type=f32) + bb2
                     + jnp.dot(h2, whh2, preferred_element_type=f32))
            return cell(gates, c2)

        h1, c1 = step1(0, zeros, zeros)

        def fused(t, carry):
            h1, c1, h2, c2, acc = carry
            h1n, c1n = step1(t, h1, c1)
            h2n, c2n = step2(h1, h2, c2)
            acc = acc + jnp.dot(h2n, wfc_ref[t - 1],
                                preferred_element_type=f32)
            return (h1n, c1n, h2n, c2n, acc)

        h1, c1, h2, c2, acc = lax.fori_loop(
            1, T, fused, (h1, c1, zeros, zeros, jnp.zeros((bh, 2 * NCP), f32)),
            unroll=2)

        h2, c2 = step2(h1, h2, c2)
        acc = acc + jnp.dot(h2, wfc_ref[T - 1], preferred_element_type=f32)

        for s in range(2):
            logits = acc[:, s * NCP:(s + 1) * NCP] + bfc
            m = jnp.max(logits, axis=1, keepdims=True)
            e = jnp.exp(logits - m)
            out_ref[0, pl.ds(s * bh, bh), :] = e / jnp.sum(e, axis=1,
                                                           keepdims=True)

    return body


def _reorder(a):
    return jnp.concatenate([a[0:2 * H], a[3 * H:4 * H], a[2 * H:3 * H]], axis=0)


def _blockdiag2(w):
    z = jnp.zeros_like(w)
    return jnp.concatenate([jnp.concatenate([w, z], 1),
                            jnp.concatenate([z, w], 1)], 0)


def _pack2_gates(w):
    w2 = _blockdiag2(w)
    k2 = w2.shape[0]
    return (w2.reshape(k2, 2, 4, H).transpose(0, 2, 1, 3).reshape(k2, 8 * H))


def _pack2_bias(bvec):
    b4 = bvec.reshape(4, 1, H)
    return jnp.broadcast_to(b4, (4, 2, H)).reshape(1, 8 * H)


def kernel(conv_w, conv_b, w_ih1, w_hh1, b_ih1, b_hh1,
           w_ih2, w_hh2, b_ih2, b_hh2, fc_w, fc_b, x):
    f32 = jnp.float32
    bf16 = jnp.bfloat16
    B = x.shape[0]
    b = min(B_TILE, 16 * pl.cdiv(B, 16))
    bh = b // 2
    G = pl.cdiv(B, b)
    B_pad = G * b

    xp = jnp.pad(x.astype(f32), ((0, B_pad - B), (0, 0), (0, 0)))
    cols = jnp.stack([xp[:, :, k:k + T_CONV] for k in range(KW)], axis=-1)
    patches = cols.transpose(0, 2, 1, 3).reshape(B_pad, T_CONV, CK)
    pa = patches.reshape(G, 2, bh, T_CONV, CK)
    pe = pa[:, :, :, 0:2 * T:2, :]
    po = pa[:, :, :, 1:2 * T:2, :]

    def mk(ps):
        return ps.transpose(0, 1, 4, 3, 2).reshape(G, CK2, T * bh)

    p_all = jnp.concatenate([mk(pe), mk(po)], axis=2).astype(bf16)

    wc = conv_w.transpose(1, 2, 0).reshape(CK, H)
    wconv = _blockdiag2(wc).astype(bf16)
    bconv = jnp.concatenate([conv_b, conv_b]).reshape(1, H2).astype(f32)
    wih1 = _pack2_gates(_reorder(w_ih1).T)
    whh1 = _pack2_gates(_reorder(w_hh1).T)
    b1 = _pack2_bias(_reorder(b_ih1 + b_hh1))
    wih2 = _pack2_gates(_reorder(w_ih2).T)
    whh2 = _pack2_gates(_reorder(w_hh2).T)
    b2 = _pack2_bias(_reorder(b_ih2 + b_hh2))
    wfc = fc_w.reshape(NC, T, H).transpose(1, 2, 0)
    wfc = jnp.pad(wfc, ((0, 0), (0, 0), (0, NCP - NC)))
    wfc_p = jax.vmap(_blockdiag2)(wfc)
    bfc = jnp.concatenate(
        [fc_b.astype(f32), jnp.full((NCP - NC,), -1e30, f32)]).reshape(1, NCP)

    def full_spec(a):
        n = a.ndim
        return pl.BlockSpec(a.shape, lambda g, n=n: (0,) * n)

    grid_spec = pltpu.PrefetchScalarGridSpec(
        num_scalar_prefetch=0,
        grid=(G,),
        in_specs=[
            pl.BlockSpec((1, CK2, 2 * T * bh), lambda g: (g, 0, 0)),
            full_spec(wconv), full_spec(bconv),
            full_spec(wih1), full_spec(whh1), full_spec(b1),
            full_spec(wih2), full_spec(whh2), full_spec(b2),
            full_spec(wfc_p), full_spec(bfc),
        ],
        out_specs=pl.BlockSpec((1, b, NCP), lambda g: (g, 0, 0)),
        scratch_shapes=[
            pltpu.VMEM((T * bh, H2), f32),
        ],
    )

    out = pl.pallas_call(
        _make_body(bh),
        out_shape=jax.ShapeDtypeStruct((G, b, NCP), f32),
        grid_spec=grid_spec,
        compiler_params=pltpu.CompilerParams(dimension_semantics=("parallel",)),
    )(p_all, wconv, bconv, wih1, whh1, b1, wih2, whh2, b2, wfc_p, bfc)
    return out.reshape(B_pad, NCP)[:B, :NC]


# speedup vs baseline: 17.7864x; 2.6890x over previous
import jax
import jax.numpy as jnp
from jax import lax
from jax.experimental import pallas as pl
from jax.experimental.pallas import tpu as pltpu

H = 64
H2 = 2 * H
KW = 8
C_IN = 2
L_IN = 128
XW = C_IN * L_IN
T_CONV = L_IN - KW + 1
T = T_CONV // 2
NC = 11
NCP = 128
B_TILE = 1024


def _make_body(bh):

    def body(x_ref, wconv_ref, bconv_ref, wih1_ref, whh1_ref, b1_ref,
             wih2_ref, whh2_ref, b2_ref, wfc_ref, bfc_ref,
             out_ref, pooled_ref):
        f32 = jnp.float32

        xw = x_ref[0]
        bconv = bconv_ref[...]

        def conv_step(t, _):
            d = jnp.dot(xw, wconv_ref[t], preferred_element_type=f32)
            ce = jnp.maximum(d[:, 0:H2] + bconv, 0.0)
            co = jnp.maximum(d[:, H2:2 * H2] + bconv, 0.0)
            pooled_ref[pl.ds(t * bh, bh), :] = jnp.maximum(ce, co)
            return 0

        lax.fori_loop(0, T, conv_step, 0, unroll=4)

        wih1 = wih1_ref[...]
        whh1 = whh1_ref[...]
        bb1 = b1_ref[...]
        wih2 = wih2_ref[...]
        whh2 = whh2_ref[...]
        bb2 = b2_ref[...]
        bfc = bfc_ref[...]
        zeros = jnp.zeros((bh, H2), f32)

        def cell(gates, c):
            sig = jax.nn.sigmoid(gates[:, 0:3 * H2])
            g = jnp.tanh(gates[:, 3 * H2:4 * H2])
            c = sig[:, H2:2 * H2] * c + sig[:, 0:H2] * g
            h = sig[:, 2 * H2:3 * H2] * jnp.tanh(c)
            return h, c

        def step1(t, h1, c1):
            r0 = pl.multiple_of(t * bh, bh)
            gates = (jnp.dot(pooled_ref[pl.ds(r0, bh), :], wih1,
                             preferred_element_type=f32) + bb1
                     + jnp.dot(h1, whh1, preferred_element_type=f32))
            return cell(gates, c1)

        def step2(h1_prev, h2, c2):
            gates = (jnp.dot(h1_prev, wih2, preferred_element_type=f32) + bb2
                     + jnp.dot(h2, whh2, preferred_element_type=f32))
            return cell(gates, c2)

        h1, c1 = step1(0, zeros, zeros)

        def fused(t, carry):
            h1, c1, h2, c2, acc = carry
            h1n, c1n = step1(t, h1, c1)
            h2n, c2n = step2(h1, h2, c2)
            acc = acc + jnp.dot(h2n, wfc_ref[t - 1],
                                preferred_element_type=f32)
            return (h1n, c1n, h2n, c2n, acc)

        h1, c1, h2, c2, acc = lax.fori_loop(
            1, T, fused, (h1, c1, zeros, zeros, jnp.zeros((bh, 2 * NCP), f32)),
            unroll=2)

        h2, c2 = step2(h1, h2, c2)
        acc = acc + jnp.dot(h2, wfc_ref[T - 1], preferred_element_type=f32)

        for s in range(2):
            logits = acc[:, s * NCP:(s + 1) * NCP] + bfc
            m = jnp.max(logits, axis=1, keepdims=True)
            e = jnp.exp(logits - m)
            out_ref[0, pl.ds(s * bh, bh), :] = e / jnp.sum(e, axis=1,
                                                           keepdims=True)

    return body


def _reorder(a):
    return jnp.concatenate([a[0:2 * H], a[3 * H:4 * H], a[2 * H:3 * H]], axis=0)


def _blockdiag2(w):
    z = jnp.zeros_like(w)
    return jnp.concatenate([jnp.concatenate([w, z], 1),
                            jnp.concatenate([z, w], 1)], 0)


def _pack2_gates(w):
    w2 = _blockdiag2(w)
    k2 = w2.shape[0]
    return (w2.reshape(k2, 2, 4, H).transpose(0, 2, 1, 3).reshape(k2, 8 * H))


def _pack2_bias(bvec):
    b4 = bvec.reshape(4, 1, H)
    return jnp.broadcast_to(b4, (4, 2, H)).reshape(1, 8 * H)


def _conv_weights(conv_w):
    f32 = jnp.float32
    wk = conv_w.transpose(1, 2, 0).astype(f32)
    taus = jnp.arange(L_IN)
    t_idx = jnp.arange(T)
    wt = jnp.zeros((T, C_IN, L_IN, 2, H), f32)
    for k in range(KW):
        for eo in range(2):
            oh = (taus[None, :] == (2 * t_idx[:, None] + eo + k)).astype(f32)
            contrib = oh[:, None, :, None] * wk[None, :, k, None, :]
            wt = wt.at[:, :, :, eo, :].add(contrib)
    wt = wt.reshape(T, XW, 2, H)
    wtp = jnp.zeros((T, 2, XW, 2, 2, H), f32)
    wtp = wtp.at[:, 0, :, :, 0, :].set(wt)
    wtp = wtp.at[:, 1, :, :, 1, :].set(wt)
    return wtp.reshape(T, 2 * XW, 2 * H2)


def kernel(conv_w, conv_b, w_ih1, w_hh1, b_ih1, b_hh1,
           w_ih2, w_hh2, b_ih2, b_hh2, fc_w, fc_b, x):
    f32 = jnp.float32
    bf16 = jnp.bfloat16
    B = x.shape[0]
    b = min(B_TILE, 16 * pl.cdiv(B, 16))
    bh = b // 2
    G = pl.cdiv(B, b)
    B_pad = G * b

    xp = jnp.pad(x.astype(f32), ((0, B_pad - B), (0, 0), (0, 0)))
    x2 = (xp.reshape(G, 2, bh, XW).transpose(0, 2, 1, 3)
          .reshape(G, bh, 2 * XW).astype(bf16))

    wconv = _conv_weights(conv_w).astype(bf16)
    bconv = jnp.concatenate([conv_b, conv_b]).reshape(1, H2).astype(f32)
    wih1 = _pack2_gates(_reorder(w_ih1).T)
    whh1 = _pack2_gates(_reorder(w_hh1).T)
    b1 = _pack2_bias(_reorder(b_ih1 + b_hh1))
    wih2 = _pack2_gates(_reorder(w_ih2).T)
    whh2 = _pack2_gates(_reorder(w_hh2).T)
    b2 = _pack2_bias(_reorder(b_ih2 + b_hh2))
    wfc = fc_w.reshape(NC, T, H).transpose(1, 2, 0)
    wfc = jnp.pad(wfc, ((0, 0), (0, 0), (0, NCP - NC)))
    wfc_p = jax.vmap(_blockdiag2)(wfc)
    bfc = jnp.concatenate(
        [fc_b.astype(f32), jnp.full((NCP - NC,), -1e30, f32)]).reshape(1, NCP)

    def full_spec(a):
        n = a.ndim
        return pl.BlockSpec(a.shape, lambda g, n=n: (0,) * n)

    grid_spec = pltpu.PrefetchScalarGridSpec(
        num_scalar_prefetch=0,
        grid=(G,),
        in_specs=[
            pl.BlockSpec((1, bh, 2 * XW), lambda g: (g, 0, 0)),
            full_spec(wconv), full_spec(bconv),
            full_spec(wih1), full_spec(whh1), full_spec(b1),
            full_spec(wih2), full_spec(whh2), full_spec(b2),
            full_spec(wfc_p), full_spec(bfc),
        ],
        out_specs=pl.BlockSpec((1, b, NCP), lambda g: (g, 0, 0)),
        scratch_shapes=[
            pltpu.VMEM((T * bh, H2), f32),
        ],
    )

    out = pl.pallas_call(
        _make_body(bh),
        out_shape=jax.ShapeDtypeStruct((G, b, NCP), f32),
        grid_spec=grid_spec,
        compiler_params=pltpu.CompilerParams(dimension_semantics=("parallel",)),
    )(x2, wconv, bconv, wih1, whh1, b1, wih2, whh2, b2, wfc_p, bfc)
    return out.reshape(B_pad, NCP)[:B, :NC]


# speedup vs baseline: 21.3126x; 1.1983x over previous
import jax
import jax.numpy as jnp
from jax import lax
from jax.experimental import pallas as pl
from jax.experimental.pallas import tpu as pltpu

H = 64
H2 = 2 * H
KW = 8
C_IN = 2
L_IN = 128
XW = C_IN * L_IN
T_CONV = L_IN - KW + 1
T = T_CONV // 2
NC = 11
NCP = 128
B_TILE = 1024


def _make_body(bh):

    def body(x_ref, wconv_ref, bconv_ref, w1_ref, b1_ref,
             w2_ref, b2_ref, wfc_ref, bfc_ref,
             out_ref, pooled_ref):
        f32 = jnp.float32

        xw = x_ref[0]
        bconv = bconv_ref[...]

        def conv_step(t, _):
            d = jnp.dot(xw, wconv_ref[t], preferred_element_type=f32)
            ce = jnp.maximum(d[:, 0:H2] + bconv, 0.0)
            co = jnp.maximum(d[:, H2:2 * H2] + bconv, 0.0)
            pooled_ref[pl.ds(t * bh, bh), :] = jnp.maximum(ce, co)
            return 0

        lax.fori_loop(0, T, conv_step, 0, unroll=4)

        w1 = w1_ref[...]
        bb1 = b1_ref[...]
        w2 = w2_ref[...]
        bb2 = b2_ref[...]
        bfc = bfc_ref[...]
        zeros = jnp.zeros((bh, H2), f32)

        def cell(gates, c):
            sig = jax.nn.sigmoid(gates[:, 0:3 * H2])
            g = jnp.tanh(gates[:, 3 * H2:4 * H2])
            c = sig[:, H2:2 * H2] * c + sig[:, 0:H2] * g
            h = sig[:, 2 * H2:3 * H2] * jnp.tanh(c)
            return h, c

        def step1(t, h1, c1):
            r0 = pl.multiple_of(t * bh, bh)
            a = jnp.concatenate([pooled_ref[pl.ds(r0, bh), :], h1], axis=1)
            gates = jnp.dot(a, w1, preferred_element_type=f32) + bb1
            return cell(gates, c1)

        def step2(h1_prev, h2, c2):
            a = jnp.concatenate([h1_prev, h2], axis=1)
            gates = jnp.dot(a, w2, preferred_element_type=f32) + bb2
            return cell(gates, c2)

        h1, c1 = step1(0, zeros, zeros)

        def fused(t, carry):
            h1, c1, h2, c2, acc = carry
            h1n, c1n = step1(t, h1, c1)
            h2n, c2n = step2(h1, h2, c2)
            acc = acc + jnp.dot(h2n, wfc_ref[t - 1],
                                preferred_element_type=f32)
            return (h1n, c1n, h2n, c2n, acc)

        h1, c1, h2, c2, acc = lax.fori_loop(
            1, T, fused, (h1, c1, zeros, zeros, jnp.zeros((bh, 2 * NCP), f32)),
            unroll=4)

        h2, c2 = step2(h1, h2, c2)
        acc = acc + jnp.dot(h2, wfc_ref[T - 1], preferred_element_type=f32)

        for s in range(2):
            logits = acc[:, s * NCP:(s + 1) * NCP] + bfc
            m = jnp.max(logits, axis=1, keepdims=True)
            e = jnp.exp(logits - m)
            out_ref[0, pl.ds(s * bh, bh), :] = e / jnp.sum(e, axis=1,
                                                           keepdims=True)

    return body


def _reorder(a):
    return jnp.concatenate([a[0:2 * H], a[3 * H:4 * H], a[2 * H:3 * H]], axis=0)


def _blockdiag2(w):
    z = jnp.zeros_like(w)
    return jnp.concatenate([jnp.concatenate([w, z], 1),
                            jnp.concatenate([z, w], 1)], 0)


def _pack2_gates(w):
    w2 = _blockdiag2(w)
    k2 = w2.shape[0]
    return (w2.reshape(k2, 2, 4, H).transpose(0, 2, 1, 3).reshape(k2, 8 * H))


def _pack2_bias(bvec):
    b4 = bvec.reshape(4, 1, H)
    return jnp.broadcast_to(b4, (4, 2, H)).reshape(1, 8 * H)


def _conv_weights(conv_w):
    f32 = jnp.float32
    wk = conv_w.transpose(1, 2, 0).astype(f32)
    taus = jnp.arange(L_IN)
    t_idx = jnp.arange(T)
    wt = jnp.zeros((T, C_IN, L_IN, 2, H), f32)
    for k in range(KW):
        for eo in range(2):
            oh = (taus[None, :] == (2 * t_idx[:, None] + eo + k)).astype(f32)
            contrib = oh[:, None, :, None] * wk[None, :, k, None, :]
            wt = wt.at[:, :, :, eo, :].add(contrib)
    wt = wt.reshape(T, XW, 2, H)
    wtp = jnp.zeros((T, 2, XW, 2, 2, H), f32)
    wtp = wtp.at[:, 0, :, :, 0, :].set(wt)
    wtp = wtp.at[:, 1, :, :, 1, :].set(wt)
    return wtp.reshape(T, 2 * XW, 2 * H2)


def kernel(conv_w, conv_b, w_ih1, w_hh1, b_ih1, b_hh1,
           w_ih2, w_hh2, b_ih2, b_hh2, fc_w, fc_b, x):
    f32 = jnp.float32
    bf16 = jnp.bfloat16
    B = x.shape[0]
    b = min(B_TILE, 16 * pl.cdiv(B, 16))
    bh = b // 2
    G = pl.cdiv(B, b)
    B_pad = G * b

    xp = jnp.pad(x.astype(f32), ((0, B_pad - B), (0, 0), (0, 0)))
    x2 = (xp.reshape(G, 2, bh, XW).transpose(0, 2, 1, 3)
          .reshape(G, bh, 2 * XW).astype(bf16))

    wconv = _conv_weights(conv_w).astype(bf16)
    bconv = jnp.concatenate([conv_b, conv_b]).reshape(1, H2).astype(f32)
    w1 = jnp.concatenate([_pack2_gates(_reorder(w_ih1).T),
                          _pack2_gates(_reorder(w_hh1).T)], axis=0)
    b1 = _pack2_bias(_reorder(b_ih1 + b_hh1))
    w2 = jnp.concatenate([_pack2_gates(_reorder(w_ih2).T),
                          _pack2_gates(_reorder(w_hh2).T)], axis=0)
    b2 = _pack2_bias(_reorder(b_ih2 + b_hh2))
    wfc = fc_w.reshape(NC, T, H).transpose(1, 2, 0)
    wfc = jnp.pad(wfc, ((0, 0), (0, 0), (0, NCP - NC)))
    wfc_p = jax.vmap(_blockdiag2)(wfc)
    bfc = jnp.concatenate(
        [fc_b.astype(f32), jnp.full((NCP - NC,), -1e30, f32)]).reshape(1, NCP)

    def full_spec(a):
        n = a.ndim
        return pl.BlockSpec(a.shape, lambda g, n=n: (0,) * n)

    grid_spec = pltpu.PrefetchScalarGridSpec(
        num_scalar_prefetch=0,
        grid=(G,),
        in_specs=[
            pl.BlockSpec((1, bh, 2 * XW), lambda g: (g, 0, 0)),
            full_spec(wconv), full_spec(bconv),
            full_spec(w1), full_spec(b1),
            full_spec(w2), full_spec(b2),
            full_spec(wfc_p), full_spec(bfc),
        ],
        out_specs=pl.BlockSpec((1, b, NCP), lambda g: (g, 0, 0)),
        scratch_shapes=[
            pltpu.VMEM((T * bh, H2), f32),
        ],
    )

    out = pl.pallas_call(
        _make_body(bh),
        out_shape=jax.ShapeDtypeStruct((G, b, NCP), f32),
        grid_spec=grid_spec,
        compiler_params=pltpu.CompilerParams(dimension_semantics=("parallel",)),
    )(x2, wconv, bconv, w1, b1, w2, b2, wfc_p, bfc)
    return out.reshape(B_pad, NCP)[:B, :NC]


# speedup vs baseline: 23.8592x; 1.1195x over previous
import jax
import jax.numpy as jnp
from jax import lax
from jax.experimental import pallas as pl
from jax.experimental.pallas import tpu as pltpu

H = 64
H2 = 2 * H
KW = 8
C_IN = 2
L_IN = 128
XW = C_IN * L_IN
T_CONV = L_IN - KW + 1
T = T_CONV // 2
NC = 11
NCP = 128
B_TILE = 1024


def _make_body(bh):

    def body(x_ref, wconv_ref, bconv_ref, w1_ref, b1_ref,
             w2_ref, b2_ref, wfc_ref, bfc_ref,
             out_ref, pooled_ref):
        f32 = jnp.float32

        xw = x_ref[0]
        bconv = bconv_ref[...]

        def conv_step(t, _):
            d = jnp.dot(xw, wconv_ref[t], preferred_element_type=f32)
            ce = jnp.maximum(d[:, 0:H2] + bconv, 0.0)
            co = jnp.maximum(d[:, H2:2 * H2] + bconv, 0.0)
            pooled_ref[pl.ds(t * bh, bh), :] = jnp.maximum(ce, co)
            return 0

        lax.fori_loop(0, T, conv_step, 0, unroll=4)

        w1 = w1_ref[...]
        bb1 = b1_ref[...]
        w2 = w2_ref[...]
        bb2 = b2_ref[...]
        bfc = bfc_ref[...]
        zeros = jnp.zeros((bh, H2), f32)

        def cell(gates, c):
            sig = 0.5 * jnp.tanh(gates[:, 0:3 * H2]) + 0.5
            g = jnp.tanh(gates[:, 3 * H2:4 * H2])
            c = sig[:, H2:2 * H2] * c + sig[:, 0:H2] * g
            h = sig[:, 2 * H2:3 * H2] * jnp.tanh(c)
            return h, c

        def step1(t, h1, c1):
            r0 = pl.multiple_of(t * bh, bh)
            a = jnp.concatenate([pooled_ref[pl.ds(r0, bh), :], h1], axis=1)
            gates = jnp.dot(a, w1, preferred_element_type=f32) + bb1
            return cell(gates, c1)

        def step2(h1_prev, h2, c2):
            a = jnp.concatenate([h1_prev, h2], axis=1)
            gates = jnp.dot(a, w2, preferred_element_type=f32) + bb2
            return cell(gates, c2)

        h1, c1 = step1(0, zeros, zeros)

        def fused(t, carry):
            h1, c1, h2, c2, acc = carry
            h1n, c1n = step1(t, h1, c1)
            h2n, c2n = step2(h1, h2, c2)
            acc = acc + jnp.dot(h2n, wfc_ref[t - 1],
                                preferred_element_type=f32)
            return (h1n, c1n, h2n, c2n, acc)

        h1, c1, h2, c2, acc = lax.fori_loop(
            1, T, fused, (h1, c1, zeros, zeros, jnp.zeros((bh, 2 * NCP), f32)),
            unroll=4)

        h2, c2 = step2(h1, h2, c2)
        acc = acc + jnp.dot(h2, wfc_ref[T - 1], preferred_element_type=f32)

        for s in range(2):
            logits = acc[:, s * NCP:(s + 1) * NCP] + bfc
            m = jnp.max(logits, axis=1, keepdims=True)
            e = jnp.exp(logits - m)
            out_ref[0, pl.ds(s * bh, bh), :] = e / jnp.sum(e, axis=1,
                                                           keepdims=True)

    return body


def _reorder(a):
    return jnp.concatenate([a[0:2 * H], a[3 * H:4 * H], a[2 * H:3 * H]], axis=0)


def _blockdiag2(w):
    z = jnp.zeros_like(w)
    return jnp.concatenate([jnp.concatenate([w, z], 1),
                            jnp.concatenate([z, w], 1)], 0)


def _pack2_gates(w):
    w2 = _blockdiag2(w)
    k2 = w2.shape[0]
    return (w2.reshape(k2, 2, 4, H).transpose(0, 2, 1, 3).reshape(k2, 8 * H))


def _pack2_bias(bvec):
    b4 = bvec.reshape(4, 1, H)
    return jnp.broadcast_to(b4, (4, 2, H)).reshape(1, 8 * H)


def _conv_weights(conv_w):
    f32 = jnp.float32
    wk = conv_w.transpose(1, 2, 0).astype(f32)
    taus = jnp.arange(L_IN)
    t_idx = jnp.arange(T)
    wt = jnp.zeros((T, C_IN, L_IN, 2, H), f32)
    for k in range(KW):
        for eo in range(2):
            oh = (taus[None, :] == (2 * t_idx[:, None] + eo + k)).astype(f32)
            contrib = oh[:, None, :, None] * wk[None, :, k, None, :]
            wt = wt.at[:, :, :, eo, :].add(contrib)
    wt = wt.reshape(T, XW, 2, H)
    wtp = jnp.zeros((T, 2, XW, 2, 2, H), f32)
    wtp = wtp.at[:, 0, :, :, 0, :].set(wt)
    wtp = wtp.at[:, 1, :, :, 1, :].set(wt)
    return wtp.reshape(T, 2 * XW, 2 * H2)


def kernel(conv_w, conv_b, w_ih1, w_hh1, b_ih1, b_hh1,
           w_ih2, w_hh2, b_ih2, b_hh2, fc_w, fc_b, x):
    f32 = jnp.float32
    bf16 = jnp.bfloat16
    B = x.shape[0]
    b = min(B_TILE, 16 * pl.cdiv(B, 16))
    bh = b // 2
    G = pl.cdiv(B, b)
    B_pad = G * b

    xp = jnp.pad(x.astype(f32), ((0, B_pad - B), (0, 0), (0, 0)))
    x2 = (xp.reshape(G, 2, bh, XW).transpose(0, 2, 1, 3)
          .reshape(G, bh, 2 * XW).astype(bf16))

    wconv = _conv_weights(conv_w).astype(bf16)
    bconv = jnp.concatenate([conv_b, conv_b]).reshape(1, H2).astype(f32)
    gsc = jnp.concatenate([jnp.full((1, 3 * H2), 0.5, f32),
                           jnp.ones((1, H2), f32)], axis=1)
    w1 = gsc * jnp.concatenate([_pack2_gates(_reorder(w_ih1).T),
                                _pack2_gates(_reorder(w_hh1).T)], axis=0)
    b1 = gsc * _pack2_bias(_reorder(b_ih1 + b_hh1))
    w2 = gsc * jnp.concatenate([_pack2_gates(_reorder(w_ih2).T),
                                _pack2_gates(_reorder(w_hh2).T)], axis=0)
    b2 = gsc * _pack2_bias(_reorder(b_ih2 + b_hh2))
    wfc = fc_w.reshape(NC, T, H).transpose(1, 2, 0)
    wfc = jnp.pad(wfc, ((0, 0), (0, 0), (0, NCP - NC)))
    wfc_p = jax.vmap(_blockdiag2)(wfc)
    bfc = jnp.concatenate(
        [fc_b.astype(f32), jnp.full((NCP - NC,), -1e30, f32)]).reshape(1, NCP)

    def full_spec(a):
        n = a.ndim
        return pl.BlockSpec(a.shape, lambda g, n=n: (0,) * n)

    grid_spec = pltpu.PrefetchScalarGridSpec(
        num_scalar_prefetch=0,
        grid=(G,),
        in_specs=[
            pl.BlockSpec((1, bh, 2 * XW), lambda g: (g, 0, 0)),
            full_spec(wconv), full_spec(bconv),
            full_spec(w1), full_spec(b1),
            full_spec(w2), full_spec(b2),
            full_spec(wfc_p), full_spec(bfc),
        ],
        out_specs=pl.BlockSpec((1, b, NCP), lambda g: (g, 0, 0)),
        scratch_shapes=[
            pltpu.VMEM((T * bh, H2), f32),
        ],
    )

    out = pl.pallas_call(
        _make_body(bh),
        out_shape=jax.ShapeDtypeStruct((G, b, NCP), f32),
        grid_spec=grid_spec,
        compiler_params=pltpu.CompilerParams(dimension_semantics=("parallel",)),
    )(x2, wconv, bconv, w1, b1, w2, b2, wfc_p, bfc)
    return out.reshape(B_pad, NCP)[:B, :NC]
